# baseline (speedup 1.0000x reference)
import functools
import math

import jax
import jax.numpy as jnp
from jax import lax
from jax.experimental import pallas as pl
from jax.experimental.pallas import tpu as pltpu
from jax.experimental.pallas import tpu_sc as plsc

_N = 10000
_E = 320000
_D = 128
_NG = 64
_BN_EPS = 1e-5
_SLOPE = 0.01

_CHUNK = 128
_EPT = _E // 32
_CPT = 80
_HALF = _CPT // 2
_NCHUNKS = 32 * _CPT
_PPT = _CPT * _CHUNK - _EPT
_NPAD = 8
_ROWS_A = 624
_NBUF = 2


def _agg_body(h_hbm, src_hbm, dst_hbm, out_hbm, idx_s, idx_d, bufs, zbuf,
              acc, sem0, sem1):
    sems = (sem0, sem1)
    cid = lax.axis_index("c")
    sid = lax.axis_index("s")
    wid = cid * 16 + sid

    def zstore(i, carry):
        r = i // 8
        c = (i % 8) * 16
        zbuf[r, pl.ds(c, 16)] = jnp.zeros((16,), jnp.float32)
        return carry

    lax.fori_loop(0, 64, zstore, 0)

    row0 = sid * _ROWS_A
    nz = jnp.where(sid == 15, 40, 39)

    def zcopy(j, carry):
        pltpu.sync_copy(zbuf, acc.at[pl.ds(row0 + j * 8, 8)])
        return carry

    lax.fori_loop(0, 2 * nz, zcopy, 0)
    plsc.subcore_barrier()

    def fire(g, b):
        pltpu.async_copy(h_hbm.at[idx_s.at[g]], bufs.at[b], sems[b])

    def drain_and_scatter(g, b):
        pltpu.make_async_copy(h_hbm.at[idx_s.at[g]], bufs.at[b],
                              sems[b]).wait()
        pltpu.sync_copy(bufs.at[b], acc.at[idx_d.at[g]], add=True)

    for half in range(2):
        c0 = wid * _CPT + half * _HALF
        pltpu.sync_copy(src_hbm.at[pl.ds(c0, _HALF)], idx_s)
        pltpu.sync_copy(dst_hbm.at[pl.ds(c0, _HALF)], idx_d)

        for b in range(_NBUF):
            fire(b, b)

        def estep(i, carry):
            g0 = i * _NBUF
            for b in range(_NBUF):
                drain_and_scatter(g0 + b, b)
                fire(g0 + b + _NBUF, b)
            return carry

        lax.fori_loop(0, _HALF // _NBUF - 1, estep, 0)
        for b in range(_NBUF):
            drain_and_scatter(_HALF - _NBUF + b, b)

    plsc.subcore_barrier()

    @pl.when(sid != 15)
    def _():
        pltpu.sync_copy(
            acc.at[pl.ds(row0, _ROWS_A)],
            out_hbm.at[pl.ds(cid * _N + row0, _ROWS_A)],
        )

    @pl.when(sid == 15)
    def _():
        pltpu.sync_copy(
            acc.at[pl.ds(row0, _N - 15 * _ROWS_A)],
            out_hbm.at[pl.ds(cid * _N + row0, _N - 15 * _ROWS_A)],
        )


@jax.jit
def _edge_agg(h, src_c, dst_c):
    mesh = plsc.VectorSubcoreMesh(core_axis_name="c", subcore_axis_name="s")
    fn = pl.kernel(
        _agg_body,
        mesh=mesh,
        out_type=jax.ShapeDtypeStruct((2 * _N, _D), jnp.float32),
        scratch_types=[
            pltpu.VMEM((_HALF, _CHUNK), jnp.int32),
            pltpu.VMEM((_HALF, _CHUNK), jnp.int32),
            pltpu.VMEM((_NBUF, _CHUNK, _D), jnp.float32),
            pltpu.VMEM((8, _D), jnp.float32),
            pltpu.VMEM_SHARED((_N, _D), jnp.float32),
            pltpu.SemaphoreType.DMA,
            pltpu.SemaphoreType.DMA,
        ],
    )
    return fn(h, src_c, dst_c)


_BNF = 1.0 / math.sqrt(1.0 + _BN_EPS)


def _conv_tc_body(h_ref, agg_ref, w_ref, b_ref, g_ref, bt_ref, ep_ref, o_ref):
    a = agg_ref[0:_N, :] + agg_ref[_N:2 * _N, :]
    x2 = (1.0 + ep_ref[...]) * h_ref[0:_N, :] + a
    t = jnp.dot(x2, w_ref[...], preferred_element_type=jnp.float32)
    t = (t + b_ref[...]) * (g_ref[...] * _BNF) + bt_ref[...]
    o_ref[0:_N, :] = jnp.where(t >= 0.0, t, t * (_SLOPE * _SLOPE))
    o_ref[_N:_N + _NPAD, :] = jnp.zeros((_NPAD, _D), jnp.float32)


@jax.jit
def _conv_update(h, agg2, w, b, gamma, beta, epsv):
    return pl.pallas_call(
        _conv_tc_body,
        out_shape=jax.ShapeDtypeStruct((_N + _NPAD, _D), jnp.float32),
    )(h, agg2, w, b, gamma, beta, epsv)


def _head_body(g2_ref, g3_ref, g4_ref, bat_ref, w1_ref, b1_ref, w2_ref,
               b2_ref, w3_ref, b3_ref, wf_ref, bf_ref, o_ref):
    bat = bat_ref[...]
    gi = lax.broadcasted_iota(jnp.int32, (_NG, _N), 0)
    oh = jnp.where(gi == bat, 1.0, 0.0).astype(jnp.float32)
    g4 = g4_ref[0:_N, :]
    pool = jnp.dot(oh, g4, preferred_element_type=jnp.float32)
    hp = lax.dot_general(oh, pool, (((0,), (0,)), ((), ())),
                         preferred_element_type=jnp.float32)
    w1 = w1_ref[...]
    z = jnp.dot(g2_ref[0:_N, :], w1[0:_D, :],
                preferred_element_type=jnp.float32)
    z = z + jnp.dot(g3_ref[0:_N, :], w1[_D:2 * _D, :],
                    preferred_element_type=jnp.float32)
    z = z + jnp.dot(g4, w1[2 * _D:3 * _D, :],
                    preferred_element_type=jnp.float32)
    z = z + jnp.dot(hp, w1[3 * _D:4 * _D, :],
                    preferred_element_type=jnp.float32)
    z = z + b1_ref[...]
    z = jnp.dot(z, w2_ref[...], preferred_element_type=jnp.float32) + b2_ref[...]
    z = jnp.where(z >= 0.0, z, z * _SLOPE)
    z = jnp.dot(z, w3_ref[...], preferred_element_type=jnp.float32) + b3_ref[...]
    z = jnp.where(z >= 0.0, z, z * _SLOPE)
    z = jnp.dot(z, wf_ref[...], preferred_element_type=jnp.float32) + bf_ref[...]
    o_ref[...] = 1.0 / (1.0 + jnp.exp(-z))


@jax.jit
def _head(g2, g3, g4, bati, w1, b1, w2, b2, w3, b3, wfp, bfp):
    return pl.pallas_call(
        _head_body,
        out_shape=jax.ShapeDtypeStruct((_N, _D), jnp.float32),
    )(g2, g3, g4, bati, w1, b1, w2, b2, w3, b3, wfp, bfp)


def kernel(x, edge_index, batch, params):
    src_n = edge_index[0].astype(jnp.int32).reshape(32, _EPT)
    dst_n = edge_index[1].astype(jnp.int32).reshape(32, _EPT)
    src_c = jnp.concatenate(
        [src_n, jnp.full((32, _PPT), _N, jnp.int32)],
        axis=1).reshape(_NCHUNKS, _CHUNK)
    pad_dst = (jnp.arange(32, dtype=jnp.int32)[:, None] * _PPT
               + jnp.arange(_PPT, dtype=jnp.int32)[None, :])
    dst_c = jnp.concatenate([dst_n, pad_dst],
                            axis=1).reshape(_NCHUNKS, _CHUNK)
    bati = batch.astype(jnp.int32).reshape(1, _N)

    def conv_params(p):
        return (p['W'], p['b'].reshape(1, _D), p['gamma'].reshape(1, _D),
                p['beta'].reshape(1, _D),
                jnp.broadcast_to(p['eps'].reshape(1, 1), (1, _D)))

    h = jnp.pad(x, ((0, _NPAD), (0, 0)))
    hs = []
    for i, p in enumerate([params['conv1']] + list(params['convs'])):
        agg2 = _edge_agg(h, src_c, dst_c)
        w, b, gamma, beta, epsv = conv_params(p)
        h = _conv_update(h, agg2, w, b, gamma, beta, epsv)
        if i > 0:
            hs.append(h)

    wfp = jnp.pad(params['final']['W'], ((0, 0), (0, _D - 1)))
    bfp = jnp.pad(params['final']['b'], (0, _D - 1)).reshape(1, _D)
    out = _head(
        hs[0], hs[1], hs[2], bati,
        params['cls1']['W'], params['cls1']['b'].reshape(1, _D),
        params['cls'][0]['W'], params['cls'][0]['b'].reshape(1, _D),
        params['cls'][1]['W'], params['cls'][1]['b'].reshape(1, _D),
        wfp, bfp,
    )
    return out[:, :1]


# speedup vs baseline: 3.0439x; 3.0439x over previous
import functools
import math

import jax
import jax.numpy as jnp
from jax import lax
from jax.experimental import pallas as pl
from jax.experimental.pallas import tpu as pltpu
from jax.experimental.pallas import tpu_sc as plsc

_N = 10000
_E = 320000
_D = 128
_NG = 64
_BN_EPS = 1e-5
_SLOPE = 0.01

_CHUNK = 128
_EPT = _E // 32
_CPT = 80
_HALF = _CPT // 2
_NCHUNKS = 32 * _CPT
_PPT = _CPT * _CHUNK - _EPT
_NPAD = 16 * _PPT
_ROWS_A = 624
_NBUF = 2


def _agg_body(h_hbm, src_hbm, dst_hbm, out_hbm, idx_s, idx_d, bufs, zbuf,
              acc, sem0, sem1):
    sems = (sem0, sem1)
    cid = lax.axis_index("c")
    sid = lax.axis_index("s")
    wid = cid * 16 + sid

    def zstore(i, carry):
        r = i // 8
        c = (i % 8) * 16
        zbuf[r, pl.ds(c, 16)] = jnp.zeros((16,), jnp.float32)
        return carry

    lax.fori_loop(0, 64, zstore, 0)

    row0 = sid * _ROWS_A
    nz = jnp.where(sid == 15, 40, 39)

    def zcopy(j, carry):
        pltpu.sync_copy(zbuf, acc.at[pl.ds(row0 + j * 8, 8)])
        return carry

    lax.fori_loop(0, 2 * nz, zcopy, 0)
    plsc.subcore_barrier()

    def fire(g, b):
        pltpu.async_copy(h_hbm.at[idx_s.at[g]], bufs.at[b], sems[b])

    def drain_and_scatter(g, b):
        pltpu.make_async_copy(h_hbm.at[idx_s.at[g]], bufs.at[b],
                              sems[b]).wait()
        pltpu.sync_copy(bufs.at[b], acc.at[idx_d.at[g]], add=True)

    for half in range(2):
        c0 = wid * _CPT + half * _HALF
        pltpu.sync_copy(src_hbm.at[pl.ds(c0, _HALF)], idx_s)
        pltpu.sync_copy(dst_hbm.at[pl.ds(c0, _HALF)], idx_d)

        for b in range(_NBUF):
            fire(b, b)

        def estep(i, carry):
            g0 = i * _NBUF
            for b in range(_NBUF):
                drain_and_scatter(g0 + b, b)
                fire(g0 + b + _NBUF, b)
            return carry

        lax.fori_loop(0, _HALF // _NBUF - 1, estep, 0)
        for b in range(_NBUF):
            drain_and_scatter(_HALF - _NBUF + b, b)

    plsc.subcore_barrier()

    @pl.when(sid != 15)
    def _():
        pltpu.sync_copy(
            acc.at[pl.ds(row0, _ROWS_A)],
            out_hbm.at[pl.ds(cid * _N + row0, _ROWS_A)],
        )

    @pl.when(sid == 15)
    def _():
        pltpu.sync_copy(
            acc.at[pl.ds(row0, _N - 15 * _ROWS_A)],
            out_hbm.at[pl.ds(cid * _N + row0, _N - 15 * _ROWS_A)],
        )


@jax.jit
def _edge_agg(h, src_c, dst_c):
    mesh = plsc.VectorSubcoreMesh(core_axis_name="c", subcore_axis_name="s")
    fn = pl.kernel(
        _agg_body,
        mesh=mesh,
        out_type=jax.ShapeDtypeStruct((2 * _N, _D), jnp.float32),
        scratch_types=[
            pltpu.VMEM((_HALF, _CHUNK), jnp.int32),
            pltpu.VMEM((_HALF, _CHUNK), jnp.int32),
            pltpu.VMEM((_NBUF, _CHUNK, _D), jnp.float32),
            pltpu.VMEM((8, _D), jnp.float32),
            pltpu.VMEM_SHARED((_N, _D), jnp.float32),
            pltpu.SemaphoreType.DMA,
            pltpu.SemaphoreType.DMA,
        ],
    )
    return fn(h, src_c, dst_c)


_BNF = 1.0 / math.sqrt(1.0 + _BN_EPS)


def _conv_tc_body(h_ref, agg_ref, w_ref, b_ref, g_ref, bt_ref, ep_ref, o_ref):
    a = agg_ref[0:_N, :] + agg_ref[_N:2 * _N, :]
    x2 = (1.0 + ep_ref[...]) * h_ref[0:_N, :] + a
    t = jnp.dot(x2, w_ref[...], preferred_element_type=jnp.float32)
    t = (t + b_ref[...]) * (g_ref[...] * _BNF) + bt_ref[...]
    o_ref[0:_N, :] = jnp.where(t >= 0.0, t, t * (_SLOPE * _SLOPE))
    o_ref[_N:_N + _NPAD, :] = jnp.zeros((_NPAD, _D), jnp.float32)


@jax.jit
def _conv_update(h, agg2, w, b, gamma, beta, epsv):
    return pl.pallas_call(
        _conv_tc_body,
        out_shape=jax.ShapeDtypeStruct((_N + _NPAD, _D), jnp.float32),
    )(h, agg2, w, b, gamma, beta, epsv)


def _head_body(g2_ref, g3_ref, g4_ref, bat_ref, w1_ref, b1_ref, w2_ref,
               b2_ref, w3_ref, b3_ref, wf_ref, bf_ref, o_ref):
    bat = bat_ref[...]
    gi = lax.broadcasted_iota(jnp.int32, (_NG, _N), 0)
    oh = jnp.where(gi == bat, 1.0, 0.0).astype(jnp.float32)
    g4 = g4_ref[0:_N, :]
    pool = jnp.dot(oh, g4, preferred_element_type=jnp.float32)
    hp = lax.dot_general(oh, pool, (((0,), (0,)), ((), ())),
                         preferred_element_type=jnp.float32)
    w1 = w1_ref[...]
    z = jnp.dot(g2_ref[0:_N, :], w1[0:_D, :],
                preferred_element_type=jnp.float32)
    z = z + jnp.dot(g3_ref[0:_N, :], w1[_D:2 * _D, :],
                    preferred_element_type=jnp.float32)
    z = z + jnp.dot(g4, w1[2 * _D:3 * _D, :],
                    preferred_element_type=jnp.float32)
    z = z + jnp.dot(hp, w1[3 * _D:4 * _D, :],
                    preferred_element_type=jnp.float32)
    z = z + b1_ref[...]
    z = jnp.dot(z, w2_ref[...], preferred_element_type=jnp.float32) + b2_ref[...]
    z = jnp.where(z >= 0.0, z, z * _SLOPE)
    z = jnp.dot(z, w3_ref[...], preferred_element_type=jnp.float32) + b3_ref[...]
    z = jnp.where(z >= 0.0, z, z * _SLOPE)
    z = jnp.dot(z, wf_ref[...], preferred_element_type=jnp.float32) + bf_ref[...]
    o_ref[...] = 1.0 / (1.0 + jnp.exp(-z))


@jax.jit
def _head(g2, g3, g4, bati, w1, b1, w2, b2, w3, b3, wfp, bfp):
    return pl.pallas_call(
        _head_body,
        out_shape=jax.ShapeDtypeStruct((_N, _D), jnp.float32),
    )(g2, g3, g4, bati, w1, b1, w2, b2, w3, b3, wfp, bfp)


def kernel(x, edge_index, batch, params):
    src_n = edge_index[0].astype(jnp.int32).reshape(32, _EPT)
    dst_n = edge_index[1].astype(jnp.int32).reshape(32, _EPT)
    pad_src = (_N + (jnp.arange(32, dtype=jnp.int32) % 16)[:, None] * _PPT
               + jnp.arange(_PPT, dtype=jnp.int32)[None, :])
    src_c = jnp.concatenate(
        [src_n, pad_src],
        axis=1).reshape(_NCHUNKS, _CHUNK)
    pad_dst = (jnp.arange(32, dtype=jnp.int32)[:, None] * _PPT
               + jnp.arange(_PPT, dtype=jnp.int32)[None, :])
    dst_c = jnp.concatenate([dst_n, pad_dst],
                            axis=1).reshape(_NCHUNKS, _CHUNK)
    bati = batch.astype(jnp.int32).reshape(1, _N)

    def conv_params(p):
        return (p['W'], p['b'].reshape(1, _D), p['gamma'].reshape(1, _D),
                p['beta'].reshape(1, _D),
                jnp.broadcast_to(p['eps'].reshape(1, 1), (1, _D)))

    h = jnp.pad(x, ((0, _NPAD), (0, 0)))
    hs = []
    for i, p in enumerate([params['conv1']] + list(params['convs'])):
        agg2 = _edge_agg(h, src_c, dst_c)
        w, b, gamma, beta, epsv = conv_params(p)
        h = _conv_update(h, agg2, w, b, gamma, beta, epsv)
        if i > 0:
            hs.append(h)

    wfp = jnp.pad(params['final']['W'], ((0, 0), (0, _D - 1)))
    bfp = jnp.pad(params['final']['b'], (0, _D - 1)).reshape(1, _D)
    out = _head(
        hs[0], hs[1], hs[2], bati,
        params['cls1']['W'], params['cls1']['b'].reshape(1, _D),
        params['cls'][0]['W'], params['cls'][0]['b'].reshape(1, _D),
        params['cls'][1]['W'], params['cls'][1]['b'].reshape(1, _D),
        wfp, bfp,
    )
    return out[:, :1]


# speedup vs baseline: 3.1694x; 1.0412x over previous
import functools
import math

import jax
import jax.numpy as jnp
from jax import lax
from jax.experimental import pallas as pl
from jax.experimental.pallas import tpu as pltpu
from jax.experimental.pallas import tpu_sc as plsc

_N = 10000
_E = 320000
_D = 128
_NG = 64
_BN_EPS = 1e-5
_SLOPE = 0.01

_CHUNK = 64
_EPT = _E // 32
_CPT = 160
_HALF = 40
_NCHUNKS = 32 * _CPT
_PPT = _CPT * _CHUNK - _EPT
_NPAD = 16 * _PPT
_ROWS_A = 624
_NBUF = 4


def _agg_body(h_hbm, src_hbm, dst_hbm, out_hbm, idx_s, idx_d, bufs, zbuf,
              acc, sem0, sem1, sem2, sem3):
    sems = (sem0, sem1, sem2, sem3)
    cid = lax.axis_index("c")
    sid = lax.axis_index("s")
    wid = cid * 16 + sid

    def zstore(i, carry):
        r = i // 8
        c = (i % 8) * 16
        zbuf[r, pl.ds(c, 16)] = jnp.zeros((16,), jnp.float32)
        return carry

    lax.fori_loop(0, 64, zstore, 0)

    row0 = sid * _ROWS_A
    nz = jnp.where(sid == 15, 40, 39)

    def zcopy(j, carry):
        pltpu.sync_copy(zbuf, acc.at[pl.ds(row0 + j * 8, 8)])
        return carry

    lax.fori_loop(0, 2 * nz, zcopy, 0)
    plsc.subcore_barrier()

    def fire(g, b):
        pltpu.async_copy(h_hbm.at[idx_s.at[g]], bufs.at[b], sems[b])

    def drain_and_scatter(g, b):
        pltpu.make_async_copy(h_hbm.at[idx_s.at[g]], bufs.at[b],
                              sems[b]).wait()
        pltpu.sync_copy(bufs.at[b], acc.at[idx_d.at[g]], add=True)

    for half in range(_CPT // _HALF):
        c0 = wid * _CPT + half * _HALF
        pltpu.sync_copy(src_hbm.at[pl.ds(c0, _HALF)], idx_s)
        pltpu.sync_copy(dst_hbm.at[pl.ds(c0, _HALF)], idx_d)

        for b in range(_NBUF):
            fire(b, b)

        def estep(i, carry):
            g0 = i * _NBUF
            for b in range(_NBUF):
                drain_and_scatter(g0 + b, b)
                fire(g0 + b + _NBUF, b)
            return carry

        lax.fori_loop(0, _HALF // _NBUF - 1, estep, 0)
        for b in range(_NBUF):
            drain_and_scatter(_HALF - _NBUF + b, b)

    plsc.subcore_barrier()

    @pl.when(sid != 15)
    def _():
        pltpu.sync_copy(
            acc.at[pl.ds(row0, _ROWS_A)],
            out_hbm.at[pl.ds(cid * _N + row0, _ROWS_A)],
        )

    @pl.when(sid == 15)
    def _():
        pltpu.sync_copy(
            acc.at[pl.ds(row0, _N - 15 * _ROWS_A)],
            out_hbm.at[pl.ds(cid * _N + row0, _N - 15 * _ROWS_A)],
        )


@jax.jit
def _edge_agg(h, src_c, dst_c):
    mesh = plsc.VectorSubcoreMesh(core_axis_name="c", subcore_axis_name="s")
    fn = pl.kernel(
        _agg_body,
        mesh=mesh,
        out_type=jax.ShapeDtypeStruct((2 * _N, _D), jnp.float32),
        scratch_types=[
            pltpu.VMEM((_HALF, _CHUNK), jnp.int32),
            pltpu.VMEM((_HALF, _CHUNK), jnp.int32),
            pltpu.VMEM((_NBUF, _CHUNK, _D), jnp.float32),
            pltpu.VMEM((8, _D), jnp.float32),
            pltpu.VMEM_SHARED((_N, _D), jnp.float32),
            pltpu.SemaphoreType.DMA,
            pltpu.SemaphoreType.DMA,
            pltpu.SemaphoreType.DMA,
            pltpu.SemaphoreType.DMA,
        ],
    )
    return fn(h, src_c, dst_c)


_BNF = 1.0 / math.sqrt(1.0 + _BN_EPS)


def _conv_tc_body(h_ref, agg_ref, w_ref, b_ref, g_ref, bt_ref, ep_ref, o_ref):
    a = agg_ref[0:_N, :] + agg_ref[_N:2 * _N, :]
    x2 = (1.0 + ep_ref[...]) * h_ref[0:_N, :] + a
    t = jnp.dot(x2, w_ref[...], preferred_element_type=jnp.float32)
    t = (t + b_ref[...]) * (g_ref[...] * _BNF) + bt_ref[...]
    o_ref[0:_N, :] = jnp.where(t >= 0.0, t, t * (_SLOPE * _SLOPE))
    o_ref[_N:_N + _NPAD, :] = jnp.zeros((_NPAD, _D), jnp.float32)


@jax.jit
def _conv_update(h, agg2, w, b, gamma, beta, epsv):
    return pl.pallas_call(
        _conv_tc_body,
        out_shape=jax.ShapeDtypeStruct((_N + _NPAD, _D), jnp.float32),
    )(h, agg2, w, b, gamma, beta, epsv)


def _head_body(g2_ref, g3_ref, g4_ref, bat_ref, w1_ref, b1_ref, w2_ref,
               b2_ref, w3_ref, b3_ref, wf_ref, bf_ref, o_ref):
    bat = bat_ref[...]
    gi = lax.broadcasted_iota(jnp.int32, (_NG, _N), 0)
    oh = jnp.where(gi == bat, 1.0, 0.0).astype(jnp.float32)
    g4 = g4_ref[0:_N, :]
    pool = jnp.dot(oh, g4, preferred_element_type=jnp.float32)
    hp = lax.dot_general(oh, pool, (((0,), (0,)), ((), ())),
                         preferred_element_type=jnp.float32)
    w1 = w1_ref[...]
    z = jnp.dot(g2_ref[0:_N, :], w1[0:_D, :],
                preferred_element_type=jnp.float32)
    z = z + jnp.dot(g3_ref[0:_N, :], w1[_D:2 * _D, :],
                    preferred_element_type=jnp.float32)
    z = z + jnp.dot(g4, w1[2 * _D:3 * _D, :],
                    preferred_element_type=jnp.float32)
    z = z + jnp.dot(hp, w1[3 * _D:4 * _D, :],
                    preferred_element_type=jnp.float32)
    z = z + b1_ref[...]
    z = jnp.dot(z, w2_ref[...], preferred_element_type=jnp.float32) + b2_ref[...]
    z = jnp.where(z >= 0.0, z, z * _SLOPE)
    z = jnp.dot(z, w3_ref[...], preferred_element_type=jnp.float32) + b3_ref[...]
    z = jnp.where(z >= 0.0, z, z * _SLOPE)
    z = jnp.dot(z, wf_ref[...], preferred_element_type=jnp.float32) + bf_ref[...]
    o_ref[...] = 1.0 / (1.0 + jnp.exp(-z))


@jax.jit
def _head(g2, g3, g4, bati, w1, b1, w2, b2, w3, b3, wfp, bfp):
    return pl.pallas_call(
        _head_body,
        out_shape=jax.ShapeDtypeStruct((_N, _D), jnp.float32),
    )(g2, g3, g4, bati, w1, b1, w2, b2, w3, b3, wfp, bfp)


def kernel(x, edge_index, batch, params):
    src_n = edge_index[0].astype(jnp.int32).reshape(32, _EPT)
    dst_n = edge_index[1].astype(jnp.int32).reshape(32, _EPT)
    pad_src = (_N + (jnp.arange(32, dtype=jnp.int32) % 16)[:, None] * _PPT
               + jnp.arange(_PPT, dtype=jnp.int32)[None, :])
    src_c = jnp.concatenate(
        [src_n, pad_src],
        axis=1).reshape(_NCHUNKS, _CHUNK)
    pad_dst = (jnp.arange(32, dtype=jnp.int32)[:, None] * _PPT
               + jnp.arange(_PPT, dtype=jnp.int32)[None, :])
    dst_c = jnp.concatenate([dst_n, pad_dst],
                            axis=1).reshape(_NCHUNKS, _CHUNK)
    bati = batch.astype(jnp.int32).reshape(1, _N)

    def conv_params(p):
        return (p['W'], p['b'].reshape(1, _D), p['gamma'].reshape(1, _D),
                p['beta'].reshape(1, _D),
                jnp.broadcast_to(p['eps'].reshape(1, 1), (1, _D)))

    h = jnp.pad(x, ((0, _NPAD), (0, 0)))
    hs = []
    for i, p in enumerate([params['conv1']] + list(params['convs'])):
        agg2 = _edge_agg(h, src_c, dst_c)
        w, b, gamma, beta, epsv = conv_params(p)
        h = _conv_update(h, agg2, w, b, gamma, beta, epsv)
        if i > 0:
            hs.append(h)

    wfp = jnp.pad(params['final']['W'], ((0, 0), (0, _D - 1)))
    bfp = jnp.pad(params['final']['b'], (0, _D - 1)).reshape(1, _D)
    out = _head(
        hs[0], hs[1], hs[2], bati,
        params['cls1']['W'], params['cls1']['b'].reshape(1, _D),
        params['cls'][0]['W'], params['cls'][0]['b'].reshape(1, _D),
        params['cls'][1]['W'], params['cls'][1]['b'].reshape(1, _D),
        wfp, bfp,
    )
    return out[:, :1]


# speedup vs baseline: 3.2109x; 1.0131x over previous
import functools
import math

import jax
import jax.numpy as jnp
from jax import lax
from jax.experimental import pallas as pl
from jax.experimental.pallas import tpu as pltpu
from jax.experimental.pallas import tpu_sc as plsc

_N = 10000
_E = 320000
_D = 128
_NG = 64
_BN_EPS = 1e-5
_SLOPE = 0.01

_CHUNK = 64
_EPT = _E // 32
_CPT = 160
_HALF = 40
_NCHUNKS = 32 * _CPT
_PPT = _CPT * _CHUNK - _EPT
_NPAD = 16 * _PPT
_ROWS_A = 624
_NBUF = 4


def _agg_body(h_hbm, src_hbm, dst_hbm, out_hbm, idx_s, idx_d, bufs, zbuf,
              acc, sem0, sem1, sem2, sem3):
    sems = (sem0, sem1, sem2, sem3)
    cid = lax.axis_index("c")
    sid = lax.axis_index("s")
    wid = cid * 16 + sid

    def zstore(i, carry):
        r = i // 8
        c = (i % 8) * 16
        zbuf[r, pl.ds(c, 16)] = jnp.zeros((16,), jnp.float32)
        return carry

    lax.fori_loop(0, 64, zstore, 0)

    row0 = sid * _ROWS_A
    nz = jnp.where(sid == 15, 40, 39)

    def zcopy(j, carry):
        pltpu.sync_copy(zbuf, acc.at[pl.ds(row0 + j * 8, 8)])
        return carry

    lax.fori_loop(0, 2 * nz, zcopy, 0)
    plsc.subcore_barrier()

    def fire(g, b):
        pltpu.async_copy(h_hbm.at[idx_s.at[g]], bufs.at[b], sems[b])

    def drain_and_scatter(g, b):
        pltpu.make_async_copy(h_hbm.at[idx_s.at[g]], bufs.at[b],
                              sems[b]).wait()
        pltpu.sync_copy(bufs.at[b], acc.at[idx_d.at[g]], add=True)

    for half in range(_CPT // _HALF):
        c0 = wid * _CPT + half * _HALF
        pltpu.sync_copy(src_hbm.at[pl.ds(c0, _HALF)], idx_s)
        pltpu.sync_copy(dst_hbm.at[pl.ds(c0, _HALF)], idx_d)

        for b in range(_NBUF):
            fire(b, b)

        def estep(i, carry):
            g0 = i * _NBUF
            for b in range(_NBUF):
                drain_and_scatter(g0 + b, b)
                fire(g0 + b + _NBUF, b)
            return carry

        lax.fori_loop(0, _HALF // _NBUF - 1, estep, 0)
        for b in range(_NBUF):
            drain_and_scatter(_HALF - _NBUF + b, b)

    plsc.subcore_barrier()

    @pl.when(sid != 15)
    def _():
        pltpu.sync_copy(
            acc.at[pl.ds(row0, _ROWS_A)],
            out_hbm.at[pl.ds(cid * _N + row0, _ROWS_A)],
        )

    @pl.when(sid == 15)
    def _():
        pltpu.sync_copy(
            acc.at[pl.ds(row0, _N - 15 * _ROWS_A)],
            out_hbm.at[pl.ds(cid * _N + row0, _N - 15 * _ROWS_A)],
        )


@jax.jit
def _edge_agg(h, src_c, dst_c):
    mesh = plsc.VectorSubcoreMesh(core_axis_name="c", subcore_axis_name="s")
    fn = pl.kernel(
        _agg_body,
        mesh=mesh,
        out_type=jax.ShapeDtypeStruct((2 * _N, _D), jnp.float32),
        scratch_types=[
            pltpu.VMEM((_HALF, _CHUNK), jnp.int32),
            pltpu.VMEM((_HALF, _CHUNK), jnp.int32),
            pltpu.VMEM((_NBUF, _CHUNK, _D), jnp.float32),
            pltpu.VMEM((8, _D), jnp.float32),
            pltpu.VMEM_SHARED((_N, _D), jnp.float32),
            pltpu.SemaphoreType.DMA,
            pltpu.SemaphoreType.DMA,
            pltpu.SemaphoreType.DMA,
            pltpu.SemaphoreType.DMA,
        ],
    )
    return fn(h, src_c, dst_c)


_BNF = 1.0 / math.sqrt(1.0 + _BN_EPS)


def _conv_tc_body(h_ref, agg_ref, w_ref, b_ref, g_ref, bt_ref, ep_ref, o_ref):
    a = agg_ref[0:_N, :] + agg_ref[_N:2 * _N, :]
    x2 = (1.0 + ep_ref[...]) * h_ref[0:_N, :] + a
    t = jnp.dot(x2, w_ref[...], preferred_element_type=jnp.float32)
    t = (t + b_ref[...]) * (g_ref[...] * _BNF) + bt_ref[...]
    o_ref[0:_N, :] = jnp.where(t >= 0.0, t, t * (_SLOPE * _SLOPE))
    o_ref[_N:_N + _NPAD, :] = jnp.zeros((_NPAD, _D), jnp.float32)


@jax.jit
def _conv_update(h, agg2, w, b, gamma, beta, epsv):
    return pl.pallas_call(
        _conv_tc_body,
        out_shape=jax.ShapeDtypeStruct((_N + _NPAD, _D), jnp.float32),
    )(h, agg2, w, b, gamma, beta, epsv)


def _head_body(h_ref, agg_ref, w_ref, b_ref, g_ref, bt_ref, ep_ref,
               g2_ref, g3_ref, bat_ref, w1_ref, b1_ref, w2_ref,
               b2_ref, w3_ref, b3_ref, wf_ref, bf_ref, o_ref):
    a = agg_ref[0:_N, :] + agg_ref[_N:2 * _N, :]
    x2 = (1.0 + ep_ref[...]) * h_ref[0:_N, :] + a
    t = jnp.dot(x2, w_ref[...], preferred_element_type=jnp.float32)
    t = (t + b_ref[...]) * (g_ref[...] * _BNF) + bt_ref[...]
    g4 = jnp.where(t >= 0.0, t, t * (_SLOPE * _SLOPE))
    bat = bat_ref[...]
    gi = lax.broadcasted_iota(jnp.int32, (_NG, _N), 0)
    oh = jnp.where(gi == bat, 1.0, 0.0).astype(jnp.float32)
    pool = jnp.dot(oh, g4, preferred_element_type=jnp.float32)
    hp = lax.dot_general(oh, pool, (((0,), (0,)), ((), ())),
                         preferred_element_type=jnp.float32)
    w1 = w1_ref[...]
    z = jnp.dot(g2_ref[0:_N, :], w1[0:_D, :],
                preferred_element_type=jnp.float32)
    z = z + jnp.dot(g3_ref[0:_N, :], w1[_D:2 * _D, :],
                    preferred_element_type=jnp.float32)
    z = z + jnp.dot(g4, w1[2 * _D:3 * _D, :],
                    preferred_element_type=jnp.float32)
    z = z + jnp.dot(hp, w1[3 * _D:4 * _D, :],
                    preferred_element_type=jnp.float32)
    z = z + b1_ref[...]
    z = jnp.dot(z, w2_ref[...], preferred_element_type=jnp.float32) + b2_ref[...]
    z = jnp.where(z >= 0.0, z, z * _SLOPE)
    z = jnp.dot(z, w3_ref[...], preferred_element_type=jnp.float32) + b3_ref[...]
    z = jnp.where(z >= 0.0, z, z * _SLOPE)
    z = jnp.dot(z, wf_ref[...], preferred_element_type=jnp.float32) + bf_ref[...]
    o_ref[...] = 1.0 / (1.0 + jnp.exp(-z))


@jax.jit
def _head(h, agg2, w, b, gamma, beta, epsv, g2, g3, bati,
          w1, b1, w2, b2, w3, b3, wfp, bfp):
    return pl.pallas_call(
        _head_body,
        out_shape=jax.ShapeDtypeStruct((_N, _D), jnp.float32),
    )(h, agg2, w, b, gamma, beta, epsv, g2, g3, bati,
      w1, b1, w2, b2, w3, b3, wfp, bfp)


def kernel(x, edge_index, batch, params):
    src_n = edge_index[0].astype(jnp.int32).reshape(32, _EPT)
    dst_n = edge_index[1].astype(jnp.int32).reshape(32, _EPT)
    pad_src = (_N + (jnp.arange(32, dtype=jnp.int32) % 16)[:, None] * _PPT
               + jnp.arange(_PPT, dtype=jnp.int32)[None, :])
    src_c = jnp.concatenate(
        [src_n, pad_src],
        axis=1).reshape(_NCHUNKS, _CHUNK)
    pad_dst = (jnp.arange(32, dtype=jnp.int32)[:, None] * _PPT
               + jnp.arange(_PPT, dtype=jnp.int32)[None, :])
    dst_c = jnp.concatenate([dst_n, pad_dst],
                            axis=1).reshape(_NCHUNKS, _CHUNK)
    bati = batch.astype(jnp.int32).reshape(1, _N)

    def conv_params(p):
        return (p['W'], p['b'].reshape(1, _D), p['gamma'].reshape(1, _D),
                p['beta'].reshape(1, _D),
                jnp.broadcast_to(p['eps'].reshape(1, 1), (1, _D)))

    h = jnp.pad(x, ((0, _NPAD), (0, 0)))
    hs = []
    plist = [params['conv1']] + list(params['convs'])
    for i, p in enumerate(plist[:3]):
        agg2 = _edge_agg(h, src_c, dst_c)
        w, b, gamma, beta, epsv = conv_params(p)
        h = _conv_update(h, agg2, w, b, gamma, beta, epsv)
        if i > 0:
            hs.append(h)

    agg2 = _edge_agg(h, src_c, dst_c)
    w, b, gamma, beta, epsv = conv_params(plist[3])
    wfp = jnp.pad(params['final']['W'], ((0, 0), (0, _D - 1)))
    bfp = jnp.pad(params['final']['b'], (0, _D - 1)).reshape(1, _D)
    out = _head(
        h, agg2, w, b, gamma, beta, epsv,
        hs[0], hs[1], bati,
        params['cls1']['W'], params['cls1']['b'].reshape(1, _D),
        params['cls'][0]['W'], params['cls'][0]['b'].reshape(1, _D),
        params['cls'][1]['W'], params['cls'][1]['b'].reshape(1, _D),
        wfp, bfp,
    )
    return out[:, :1]
